# Initial kernel scaffold; baseline (speedup 1.0000x reference)
import functools

import jax
import jax.numpy as jnp
from jax import lax
from jax.experimental import pallas as pl
from jax.experimental.pallas import tpu as pltpu
from jax.experimental.pallas import tpu_sc as plsc

N = 10000
E = 160000
NC = 2
NS = 16
NW = NC * NS
CHUNK = 128
NCHUNK = E // CHUNK
CHUNKS_PER_TILE = (NCHUNK + NW - 1) // NW
ROWS_PER_TILE = 640
NPAD = ROWS_PER_TILE * NS

_MESH = plsc.VectorSubcoreMesh(core_axis_name="c", subcore_axis_name="s",
                               num_cores=NC, num_subcores=NS)


def _seg_sum_body(table, src, dst, out, acc, rows, idxs, idxd, sem):
    c = lax.axis_index("c")
    s = lax.axis_index("s")
    w = s * NC + c

    z16 = jnp.zeros((16,), jnp.float32)

    def zero_row(r, _):
        for j in range(8):
            rows[r, pl.ds(j * 16, 16)] = z16
        return 0

    lax.fori_loop(0, CHUNK, zero_row, 0)
    tile_r0 = pl.multiple_of(s * ROWS_PER_TILE, 128)
    for k in range(ROWS_PER_TILE // CHUNK):
        pltpu.sync_copy(rows, acc.at[pl.ds(tile_r0 + k * CHUNK, CHUNK)])
    plsc.subcore_barrier()

    def chunk_body(j, _):
        chunk = w + j * NW

        @pl.when(chunk < NCHUNK)
        def _():
            base = pl.multiple_of(chunk * CHUNK, 128)
            pltpu.sync_copy(src.at[pl.ds(base, CHUNK)], idxs)
            pltpu.sync_copy(dst.at[pl.ds(base, CHUNK)], idxd)
            pltpu.async_copy(table.at[idxs], rows, sem).wait()
            pltpu.sync_copy(rows, acc.at[idxd], add=True)

        return 0

    lax.fori_loop(0, CHUNKS_PER_TILE, chunk_body, 0)
    plsc.subcore_barrier()

    out_r0 = c * NPAD + tile_r0
    for k in range(ROWS_PER_TILE // CHUNK):
        pltpu.sync_copy(acc.at[pl.ds(tile_r0 + k * CHUNK, CHUNK)], rows)
        pltpu.sync_copy(rows, out.at[pl.ds(out_r0 + k * CHUNK, CHUNK)])


@jax.jit
def _seg_sum(table, src, dst):
    return pl.kernel(
        _seg_sum_body,
        out_type=jax.ShapeDtypeStruct((NC * NPAD, 128), jnp.float32),
        mesh=_MESH,
        scratch_types=[
            pltpu.VMEM_SHARED((NPAD, 128), jnp.float32),
            pltpu.VMEM((CHUNK, 128), jnp.float32),
            pltpu.VMEM((CHUNK,), jnp.int32),
            pltpu.VMEM((CHUNK,), jnp.int32),
            pltpu.SemaphoreType.DMA,
        ],
    )(table, src, dst)


def _deg_body(dst, out, acc, buf, idxd):
    c = lax.axis_index("c")
    s = lax.axis_index("s")
    w = s * NC + c

    z16 = jnp.zeros((16,), jnp.float32)

    def zero_row(r, _):
        buf[r, :] = z16
        return 0

    lax.fori_loop(0, CHUNK, zero_row, 0)
    tile_r0 = pl.multiple_of(s * ROWS_PER_TILE, 128)
    for k in range(ROWS_PER_TILE // CHUNK):
        pltpu.sync_copy(buf, acc.at[pl.ds(tile_r0 + k * CHUNK, CHUNK)])

    o16 = jnp.ones((16,), jnp.float32)

    def ones_row(r, _):
        buf[r, :] = o16
        return 0

    lax.fori_loop(0, CHUNK, ones_row, 0)
    plsc.subcore_barrier()

    def chunk_body(j, _):
        chunk = w + j * NW

        @pl.when(chunk < NCHUNK)
        def _():
            base = pl.multiple_of(chunk * CHUNK, 128)
            pltpu.sync_copy(dst.at[pl.ds(base, CHUNK)], idxd)
            pltpu.sync_copy(buf, acc.at[idxd], add=True)

        return 0

    lax.fori_loop(0, CHUNKS_PER_TILE, chunk_body, 0)
    plsc.subcore_barrier()

    out_r0 = c * NPAD + tile_r0
    for k in range(ROWS_PER_TILE // CHUNK):
        pltpu.sync_copy(acc.at[pl.ds(tile_r0 + k * CHUNK, CHUNK)], buf)
        pltpu.sync_copy(buf, out.at[pl.ds(out_r0 + k * CHUNK, CHUNK)])


@jax.jit
def _deg_count(dst):
    return pl.kernel(
        _deg_body,
        out_type=jax.ShapeDtypeStruct((NC * NPAD, 16), jnp.float32),
        mesh=_MESH,
        scratch_types=[
            pltpu.VMEM_SHARED((NPAD, 16), jnp.float32),
            pltpu.VMEM((CHUNK, 16), jnp.float32),
            pltpu.VMEM((CHUNK,), jnp.int32),
        ],
    )(dst)


_BN = 1000
_GRID = N // _BN


def _full(shape):
    return pl.BlockSpec(shape, lambda i: tuple(0 for _ in shape))


def _rows(shape):
    return pl.BlockSpec(shape, lambda i: (i,) + tuple(0 for _ in shape[1:]))


def _parts(shape):
    return pl.BlockSpec(shape, lambda i: (0, i, 0))


def _invdeg_body(dp_ref, out_ref):
    d = dp_ref[0] + dp_ref[1]
    out_ref[...] = 1.0 / jnp.clip(d, 1.0, None)


@jax.jit
def _invdeg(degp):
    return pl.pallas_call(
        _invdeg_body,
        grid=(_GRID,),
        in_specs=[_parts((NC, _BN, 16))],
        out_specs=_rows((_BN, 16)),
        out_shape=jax.ShapeDtypeStruct((N, 16), jnp.float32),
    )(degp)


def _dot(a, b):
    return jnp.dot(a, b, preferred_element_type=jnp.float32)


def _tc1_body(a0_ref, a1_ref, invd_ref, x_ref, w1l_ref, w1r_ref, b1_ref,
              w2l_ref, h1_ref, p2_ref):
    invd = invd_ref[:, 0:1]
    a0 = (a0_ref[0] + a0_ref[1]) * invd
    a1 = (a1_ref[0] + a1_ref[1]) * invd
    agg = jnp.concatenate([a0, a1], axis=1)
    h1 = jax.nn.relu(_dot(agg, w1l_ref[...]) + _dot(x_ref[...], w1r_ref[...])
                     + b1_ref[...])
    h1_ref[...] = h1
    p2_ref[...] = _dot(h1, w2l_ref[...])


@jax.jit
def _tc1(a0, a1, invd, x, W1l, W1r, b1, W2l):
    return pl.pallas_call(
        _tc1_body,
        grid=(_GRID,),
        in_specs=[_parts((NC, _BN, 128)), _parts((NC, _BN, 128)),
                  _rows((_BN, 16)), _rows((_BN, 256)),
                  _full((256, 256)), _full((256, 256)), _full((1, 256)),
                  _full((256, 128))],
        out_specs=[_rows((_BN, 256)), _rows((_BN, 128))],
        out_shape=[jax.ShapeDtypeStruct((N, 256), jnp.float32),
                   jax.ShapeDtypeStruct((N, 128), jnp.float32)],
    )(a0, a1, invd, x, W1l, W1r, b1, W2l)


def _tc2_body(ap_ref, invd_ref, h1_ref, w2r_ref, b2_ref, h2_ref):
    agg = (ap_ref[0] + ap_ref[1]) * invd_ref[:, 0:1]
    h2_ref[...] = jax.nn.relu(agg + _dot(h1_ref[...], w2r_ref[...])
                              + b2_ref[...])


@jax.jit
def _tc2(ap2, invd, h1, W2r, b2):
    return pl.pallas_call(
        _tc2_body,
        grid=(_GRID,),
        in_specs=[_parts((NC, _BN, 128)), _rows((_BN, 16)), _rows((_BN, 256)),
                  _full((256, 128)), _full((1, 128))],
        out_specs=_rows((_BN, 128)),
        out_shape=jax.ShapeDtypeStruct((N, 128), jnp.float32),
    )(ap2, invd, h1, W2r, b2)


def _tc3_body(ap_ref, invd_ref, h2_ref, w3l_ref, w3r_ref, b3_ref, w4l_ref,
              h3_ref, p4a_ref, p4b_ref):
    agg = (ap_ref[0] + ap_ref[1]) * invd_ref[:, 0:1]
    h3 = jax.nn.relu(_dot(agg, w3l_ref[...]) + _dot(h2_ref[...], w3r_ref[...])
                     + b3_ref[...])
    h3_ref[...] = h3
    p4 = _dot(h3, w4l_ref[...])
    p4a_ref[...] = p4[:, :128]
    p4b_ref[...] = p4[:, 128:]


@jax.jit
def _tc3(ah2, invd, h2, W3l, W3r, b3, W4l):
    return pl.pallas_call(
        _tc3_body,
        grid=(_GRID,),
        in_specs=[_parts((NC, _BN, 128)), _rows((_BN, 16)), _rows((_BN, 128)),
                  _full((128, 256)), _full((128, 256)), _full((1, 256)),
                  _full((256, 256))],
        out_specs=[_rows((_BN, 256)), _rows((_BN, 128)), _rows((_BN, 128))],
        out_shape=[jax.ShapeDtypeStruct((N, 256), jnp.float32),
                   jax.ShapeDtypeStruct((N, 128), jnp.float32),
                   jax.ShapeDtypeStruct((N, 128), jnp.float32)],
    )(ah2, invd, h2, W3l, W3r, b3, W4l)


def _tc4_body(a0_ref, a1_ref, invd_ref, h3_ref, w4r_ref, b4_ref, out_ref):
    invd = invd_ref[:, 0:1]
    a0 = (a0_ref[0] + a0_ref[1]) * invd
    a1 = (a1_ref[0] + a1_ref[1]) * invd
    agg = jnp.concatenate([a0, a1], axis=1)
    out_ref[...] = agg + _dot(h3_ref[...], w4r_ref[...]) + b4_ref[...]


@jax.jit
def _tc4(a4a, a4b, invd, h3, W4r, b4):
    return pl.pallas_call(
        _tc4_body,
        grid=(_GRID,),
        in_specs=[_parts((NC, _BN, 128)), _parts((NC, _BN, 128)),
                  _rows((_BN, 16)), _rows((_BN, 256)),
                  _full((256, 256)), _full((1, 256))],
        out_specs=_rows((_BN, 256)),
        out_shape=jax.ShapeDtypeStruct((N, 256), jnp.float32),
    )(a4a, a4b, invd, h3, W4r, b4)


def _partials(flat):
    return flat.reshape(NC, NPAD, -1)[:, :N, :]


def kernel(x, edge_index, W1l, W1r, b1, W2l, W2r, b2, W3l, W3r, b3,
           W4l, W4r, b4):
    src = edge_index[0].astype(jnp.int32)
    dst = edge_index[1].astype(jnp.int32)

    degp = _partials(_deg_count(dst))
    invd = _invdeg(degp)

    xh0 = jnp.ascontiguousarray(x[:, :128])
    xh1 = jnp.ascontiguousarray(x[:, 128:])
    a0 = _partials(_seg_sum(xh0, src, dst))
    a1 = _partials(_seg_sum(xh1, src, dst))
    h1, p2 = _tc1(a0, a1, invd, x, W1l, W1r, b1.reshape(1, -1), W2l)

    ap2 = _partials(_seg_sum(p2, src, dst))
    h2 = _tc2(ap2, invd, h1, W2r, b2.reshape(1, -1))

    ah2 = _partials(_seg_sum(h2, src, dst))
    h3, p4a, p4b = _tc3(ah2, invd, h2, W3l, W3r, b3.reshape(1, -1), W4l)

    a4a = _partials(_seg_sum(p4a, src, dst))
    a4b = _partials(_seg_sum(p4b, src, dst))
    out = _tc4(a4a, a4b, invd, h3, W4r, b4.reshape(1, -1))
    return out


# speedup vs baseline: 4.6694x; 4.6694x over previous
import functools

import jax
import jax.numpy as jnp
from jax import lax
from jax.experimental import pallas as pl
from jax.experimental.pallas import tpu as pltpu
from jax.experimental.pallas import tpu_sc as plsc

N = 10000
E = 160000
NC = 2
NS = 16
NW = NC * NS
CHUNK = 128
NCHUNK = E // CHUNK
CHUNKS_PER_TILE = (NCHUNK + NW - 1) // NW
ROWS_PER_TILE = 640
NPAD = ROWS_PER_TILE * NS

_MESH = plsc.VectorSubcoreMesh(core_axis_name="c", subcore_axis_name="s",
                               num_cores=NC, num_subcores=NS)


def _seg_sum_body(table, src, dst, out, acc, rows, idxs, idxd, sem):
    c = lax.axis_index("c")
    s = lax.axis_index("s")
    w = s * NC + c

    z16 = jnp.zeros((16,), jnp.float32)

    def zero_row(r, _):
        for j in range(8):
            rows[r, pl.ds(j * 16, 16)] = z16
        return 0

    lax.fori_loop(0, CHUNK, zero_row, 0)
    tile_r0 = pl.multiple_of(s * ROWS_PER_TILE, 128)
    for k in range(ROWS_PER_TILE // CHUNK):
        pltpu.sync_copy(rows, acc.at[pl.ds(tile_r0 + k * CHUNK, CHUNK)])
    plsc.subcore_barrier()

    def chunk_body(j, _):
        chunk = w + j * NW

        @pl.when(chunk < NCHUNK)
        def _():
            base = pl.multiple_of(chunk * CHUNK, 128)
            pltpu.sync_copy(src.at[pl.ds(base, CHUNK)], idxs)
            pltpu.sync_copy(dst.at[pl.ds(base, CHUNK)], idxd)
            pltpu.async_copy(table.at[idxs], rows, sem).wait()
            pltpu.sync_copy(rows, acc.at[idxd], add=True)

        return 0

    lax.fori_loop(0, CHUNKS_PER_TILE, chunk_body, 0)
    plsc.subcore_barrier()

    out_r0 = c * NPAD + tile_r0
    for k in range(ROWS_PER_TILE // CHUNK):
        pltpu.sync_copy(acc.at[pl.ds(tile_r0 + k * CHUNK, CHUNK)], rows)
        pltpu.sync_copy(rows, out.at[pl.ds(out_r0 + k * CHUNK, CHUNK)])


@jax.jit
def _seg_sum(table, src, dst):
    return pl.kernel(
        _seg_sum_body,
        out_type=jax.ShapeDtypeStruct((NC * NPAD, 128), jnp.float32),
        mesh=_MESH,
        scratch_types=[
            pltpu.VMEM_SHARED((NPAD, 128), jnp.float32),
            pltpu.VMEM((CHUNK, 128), jnp.float32),
            pltpu.VMEM((CHUNK,), jnp.int32),
            pltpu.VMEM((CHUNK,), jnp.int32),
            pltpu.SemaphoreType.DMA,
        ],
    )(table, src, dst)


def _deg_body(dst, out, acc, buf, idxd):
    c = lax.axis_index("c")
    s = lax.axis_index("s")
    w = s * NC + c

    z16 = jnp.zeros((16,), jnp.float32)

    def zero_row(r, _):
        for j in range(8):
            buf[r, pl.ds(j * 16, 16)] = z16
        return 0

    lax.fori_loop(0, CHUNK, zero_row, 0)
    tile_r0 = pl.multiple_of(s * ROWS_PER_TILE, 128)
    for k in range(ROWS_PER_TILE // CHUNK):
        pltpu.sync_copy(buf, acc.at[pl.ds(tile_r0 + k * CHUNK, CHUNK)])

    o16 = jnp.ones((16,), jnp.float32)

    def ones_row(r, _):
        for j in range(8):
            buf[r, pl.ds(j * 16, 16)] = o16
        return 0

    lax.fori_loop(0, CHUNK, ones_row, 0)
    plsc.subcore_barrier()

    def chunk_body(j, _):
        chunk = w + j * NW

        @pl.when(chunk < NCHUNK)
        def _():
            base = pl.multiple_of(chunk * CHUNK, 128)
            pltpu.sync_copy(dst.at[pl.ds(base, CHUNK)], idxd)
            pltpu.sync_copy(buf, acc.at[idxd], add=True)

        return 0

    lax.fori_loop(0, CHUNKS_PER_TILE, chunk_body, 0)
    plsc.subcore_barrier()

    out_r0 = c * NPAD + tile_r0
    for k in range(ROWS_PER_TILE // CHUNK):
        pltpu.sync_copy(acc.at[pl.ds(tile_r0 + k * CHUNK, CHUNK)], buf)
        pltpu.sync_copy(buf, out.at[pl.ds(out_r0 + k * CHUNK, CHUNK)])


@jax.jit
def _deg_count(dst):
    return pl.kernel(
        _deg_body,
        out_type=jax.ShapeDtypeStruct((NC * NPAD, 128), jnp.float32),
        mesh=_MESH,
        scratch_types=[
            pltpu.VMEM_SHARED((NPAD, 128), jnp.float32),
            pltpu.VMEM((CHUNK, 128), jnp.float32),
            pltpu.VMEM((CHUNK,), jnp.int32),
        ],
    )(dst)


_BN = 1000
_GRID = N // _BN


def _full(shape):
    return pl.BlockSpec(shape, lambda i: tuple(0 for _ in shape))


def _rows(shape):
    return pl.BlockSpec(shape, lambda i: (i,) + tuple(0 for _ in shape[1:]))


def _parts(shape):
    return pl.BlockSpec(shape, lambda i: (0, i, 0))


def _invdeg_body(dp_ref, out_ref):
    d = dp_ref[0] + dp_ref[1]
    out_ref[...] = (1.0 / jnp.clip(d, 1.0, None))[:, :16]


@jax.jit
def _invdeg(degp):
    return pl.pallas_call(
        _invdeg_body,
        grid=(_GRID,),
        in_specs=[_parts((NC, _BN, 128))],
        out_specs=_rows((_BN, 16)),
        out_shape=jax.ShapeDtypeStruct((N, 16), jnp.float32),
    )(degp)


def _dot(a, b):
    return jnp.dot(a, b, preferred_element_type=jnp.float32)


def _tc1_body(a0_ref, a1_ref, invd_ref, x_ref, w1l_ref, w1r_ref, b1_ref,
              w2l_ref, h1_ref, p2_ref):
    invd = invd_ref[:, 0:1]
    a0 = (a0_ref[0] + a0_ref[1]) * invd
    a1 = (a1_ref[0] + a1_ref[1]) * invd
    agg = jnp.concatenate([a0, a1], axis=1)
    h1 = jax.nn.relu(_dot(agg, w1l_ref[...]) + _dot(x_ref[...], w1r_ref[...])
                     + b1_ref[...])
    h1_ref[...] = h1
    p2_ref[...] = _dot(h1, w2l_ref[...])


@jax.jit
def _tc1(a0, a1, invd, x, W1l, W1r, b1, W2l):
    return pl.pallas_call(
        _tc1_body,
        grid=(_GRID,),
        in_specs=[_parts((NC, _BN, 128)), _parts((NC, _BN, 128)),
                  _rows((_BN, 16)), _rows((_BN, 256)),
                  _full((256, 256)), _full((256, 256)), _full((1, 256)),
                  _full((256, 128))],
        out_specs=[_rows((_BN, 256)), _rows((_BN, 128))],
        out_shape=[jax.ShapeDtypeStruct((N, 256), jnp.float32),
                   jax.ShapeDtypeStruct((N, 128), jnp.float32)],
    )(a0, a1, invd, x, W1l, W1r, b1, W2l)


def _tc2_body(ap_ref, invd_ref, h1_ref, w2r_ref, b2_ref, h2_ref):
    agg = (ap_ref[0] + ap_ref[1]) * invd_ref[:, 0:1]
    h2_ref[...] = jax.nn.relu(agg + _dot(h1_ref[...], w2r_ref[...])
                              + b2_ref[...])


@jax.jit
def _tc2(ap2, invd, h1, W2r, b2):
    return pl.pallas_call(
        _tc2_body,
        grid=(_GRID,),
        in_specs=[_parts((NC, _BN, 128)), _rows((_BN, 16)), _rows((_BN, 256)),
                  _full((256, 128)), _full((1, 128))],
        out_specs=_rows((_BN, 128)),
        out_shape=jax.ShapeDtypeStruct((N, 128), jnp.float32),
    )(ap2, invd, h1, W2r, b2)


def _tc3_body(ap_ref, invd_ref, h2_ref, w3l_ref, w3r_ref, b3_ref, w4l_ref,
              h3_ref, p4a_ref, p4b_ref):
    agg = (ap_ref[0] + ap_ref[1]) * invd_ref[:, 0:1]
    h3 = jax.nn.relu(_dot(agg, w3l_ref[...]) + _dot(h2_ref[...], w3r_ref[...])
                     + b3_ref[...])
    h3_ref[...] = h3
    p4 = _dot(h3, w4l_ref[...])
    p4a_ref[...] = p4[:, :128]
    p4b_ref[...] = p4[:, 128:]


@jax.jit
def _tc3(ah2, invd, h2, W3l, W3r, b3, W4l):
    return pl.pallas_call(
        _tc3_body,
        grid=(_GRID,),
        in_specs=[_parts((NC, _BN, 128)), _rows((_BN, 16)), _rows((_BN, 128)),
                  _full((128, 256)), _full((128, 256)), _full((1, 256)),
                  _full((256, 256))],
        out_specs=[_rows((_BN, 256)), _rows((_BN, 128)), _rows((_BN, 128))],
        out_shape=[jax.ShapeDtypeStruct((N, 256), jnp.float32),
                   jax.ShapeDtypeStruct((N, 128), jnp.float32),
                   jax.ShapeDtypeStruct((N, 128), jnp.float32)],
    )(ah2, invd, h2, W3l, W3r, b3, W4l)


def _tc4_body(a0_ref, a1_ref, invd_ref, h3_ref, w4r_ref, b4_ref, out_ref):
    invd = invd_ref[:, 0:1]
    a0 = (a0_ref[0] + a0_ref[1]) * invd
    a1 = (a1_ref[0] + a1_ref[1]) * invd
    agg = jnp.concatenate([a0, a1], axis=1)
    out_ref[...] = agg + _dot(h3_ref[...], w4r_ref[...]) + b4_ref[...]


@jax.jit
def _tc4(a4a, a4b, invd, h3, W4r, b4):
    return pl.pallas_call(
        _tc4_body,
        grid=(_GRID,),
        in_specs=[_parts((NC, _BN, 128)), _parts((NC, _BN, 128)),
                  _rows((_BN, 16)), _rows((_BN, 256)),
                  _full((256, 256)), _full((1, 256))],
        out_specs=_rows((_BN, 256)),
        out_shape=jax.ShapeDtypeStruct((N, 256), jnp.float32),
    )(a4a, a4b, invd, h3, W4r, b4)


def _partials(flat):
    return flat.reshape(NC, NPAD, -1)[:, :N, :]


def kernel(x, edge_index, W1l, W1r, b1, W2l, W2r, b2, W3l, W3r, b3,
           W4l, W4r, b4):
    src = edge_index[0].astype(jnp.int32)
    dst = edge_index[1].astype(jnp.int32)

    degp = _partials(_deg_count(dst))
    invd = _invdeg(degp)

    xh0 = x[:, :128]
    xh1 = x[:, 128:]
    a0 = _partials(_seg_sum(xh0, src, dst))
    a1 = _partials(_seg_sum(xh1, src, dst))
    h1, p2 = _tc1(a0, a1, invd, x, W1l, W1r, b1.reshape(1, -1), W2l)

    ap2 = _partials(_seg_sum(p2, src, dst))
    h2 = _tc2(ap2, invd, h1, W2r, b2.reshape(1, -1))

    ah2 = _partials(_seg_sum(h2, src, dst))
    h3, p4a, p4b = _tc3(ah2, invd, h2, W3l, W3r, b3.reshape(1, -1), W4l)

    a4a = _partials(_seg_sum(p4a, src, dst))
    a4b = _partials(_seg_sum(p4b, src, dst))
    out = _tc4(a4a, a4b, invd, h3, W4r, b4.reshape(1, -1))
    return out
